# Initial kernel scaffold; baseline (speedup 1.0000x reference)
import jax
import jax.numpy as jnp
from jax.experimental import pallas as pl


def kernel(score, delta, img):
    raise NotImplementedError("write your pallas kernel here")


# speedup vs baseline: 13.6466x; 13.6466x over previous
import numpy as np
import jax
import jax.numpy as jnp
from jax.experimental import pallas as pl
from jax.experimental.pallas import tpu as pltpu

_STRIDE = 16
_PRE_NMS_TOPN = 100
_NMS_THRESH = 0.3
_MIN_SIZE = 16.0
_H = 64
_W = 64
_A = 9
_N = _H * _W * _A
_ROWS = _N // 128
_NEG_INF = float("-inf")


def _gen_base_anchors():
    base = np.array([1.0, 1.0, 16.0, 16.0]) - 1.0
    w = base[2] - base[0] + 1.0
    h = base[3] - base[1] + 1.0
    x_ctr = base[0] + 0.5 * (w - 1.0)
    y_ctr = base[1] + 0.5 * (h - 1.0)
    size = w * h
    ratios = np.array([0.5, 1.0, 2.0])
    ws = np.round(np.sqrt(size / ratios))
    hs = np.round(ws * ratios)
    ratio_anchors = np.stack(
        [x_ctr - 0.5 * (ws - 1.0), y_ctr - 0.5 * (hs - 1.0),
         x_ctr + 0.5 * (ws - 1.0), y_ctr + 0.5 * (hs - 1.0)], axis=1)
    out = []
    scales = np.array([8.0, 16.0, 32.0])
    for i in range(ratio_anchors.shape[0]):
        a = ratio_anchors[i]
        w = a[2] - a[0] + 1.0
        h = a[3] - a[1] + 1.0
        x_ctr = a[0] + 0.5 * (w - 1.0)
        y_ctr = a[1] + 0.5 * (h - 1.0)
        ws = w * scales
        hs = h * scales
        out.append(np.stack(
            [x_ctr - 0.5 * (ws - 1.0), y_ctr - 0.5 * (hs - 1.0),
             x_ctr + 0.5 * (ws - 1.0), y_ctr + 0.5 * (hs - 1.0)], axis=1))
    return np.concatenate(out, axis=0).astype(np.float32)


def _anchor_tables():
    anchors = _gen_base_anchors()
    shifts = np.array([[i, j, i, j] for j in range(_H) for i in range(_W)],
                      dtype=np.float32) * _STRIDE
    grid = anchors[None, :, :] + shifts[:, None, :]
    flat = grid.reshape(_N, 4)
    wa = flat[:, 2] - flat[:, 0] + 1.0
    ha = flat[:, 3] - flat[:, 1] + 1.0
    cxa = flat[:, 0] + 0.5 * wa
    cya = flat[:, 1] + 0.5 * ha
    rs = lambda v: v.reshape(_ROWS, 128)
    return rs(wa), rs(ha), rs(cxa), rs(cya)


_WA, _HA, _CXA, _CYA = _anchor_tables()


def _proposal_kernel(sc_ref, dx_ref, dy_ref, dw_ref, dh_ref,
                     wa_ref, ha_ref, cx_ref, cy_ref, img_ref, out_ref,
                     masked_s, x1_s, y1_s, x2_s, y2_s, sc_s, valid_s):
    b = pl.program_id(0)
    im_h = img_ref[0, 0]
    im_w = img_ref[0, 1]
    wa = wa_ref[:]
    ha = ha_ref[:]
    cxa = cx_ref[:]
    cya = cy_ref[:]
    dx = dx_ref[0]
    dy = dy_ref[0]
    dw = dw_ref[0]
    dh = dh_ref[0]
    sc = sc_ref[0]

    pw = jnp.exp(dw) * wa
    ph = jnp.exp(dh) * ha
    pcx = dx * wa + cxa
    pcy = dy * ha + cya
    x1 = jnp.clip(pcx - 0.5 * pw, 0.0, im_w - 1.0)
    y1 = jnp.clip(pcy - 0.5 * ph, 0.0, im_h - 1.0)
    x2 = jnp.clip(pcx + 0.5 * pw, 0.0, im_w - 1.0)
    y2 = jnp.clip(pcy + 0.5 * ph, 0.0, im_h - 1.0)

    x1_s[:] = x1
    y1_s[:] = y1
    x2_s[:] = x2
    y2_s[:] = y2
    sc_s[:] = sc

    @pl.when(b == 0)
    def _():
        ws = x2 - x1 + 1.0
        hs = y2 - y1 + 1.0
        keep0 = (ws >= _MIN_SIZE) & (hs >= _MIN_SIZE)
        valid_s[:] = keep0.astype(jnp.float32)

    masked_s[:] = jnp.where(valid_s[:] > 0.5, sc, _NEG_INF)

    flat_iota = (jax.lax.broadcasted_iota(jnp.int32, (_ROWS, 128), 0) * 128
                 + jax.lax.broadcasted_iota(jnp.int32, (_ROWS, 128), 1))
    lane = jax.lax.broadcasted_iota(jnp.int32, (1, 128), 1)

    def sel_body(t, carry):
        sx1, sy1, sx2, sy2, ss, sv = carry
        mk = masked_s[:]
        m = jnp.max(mk)
        idx = jnp.min(jnp.where(mk == m, flat_iota, jnp.int32(2 ** 30)))
        r = idx // 128
        c = idx - r * 128
        lm = lane == c
        row = masked_s[pl.ds(r, 1), :]
        masked_s[pl.ds(r, 1), :] = jnp.where(lm, _NEG_INF, row)
        xv1 = jnp.sum(jnp.where(lm, x1_s[pl.ds(r, 1), :], 0.0))
        yv1 = jnp.sum(jnp.where(lm, y1_s[pl.ds(r, 1), :], 0.0))
        xv2 = jnp.sum(jnp.where(lm, x2_s[pl.ds(r, 1), :], 0.0))
        yv2 = jnp.sum(jnp.where(lm, y2_s[pl.ds(r, 1), :], 0.0))
        sval = jnp.sum(jnp.where(lm, sc_s[pl.ds(r, 1), :], 0.0))
        vval = jnp.sum(jnp.where(lm, valid_s[pl.ds(r, 1), :], 0.0))
        tm = lane == t
        return (jnp.where(tm, xv1, sx1), jnp.where(tm, yv1, sy1),
                jnp.where(tm, xv2, sx2), jnp.where(tm, yv2, sy2),
                jnp.where(tm, sval, ss), jnp.where(tm, vval, sv))

    zeros = jnp.zeros((1, 128), jnp.float32)
    sx1, sy1, sx2, sy2, ss, sv = jax.lax.fori_loop(
        0, _PRE_NMS_TOPN, sel_body, (zeros, zeros, zeros, zeros, zeros, zeros))

    areas = (sx2 - sx1 + 1.0) * (sy2 - sy1 + 1.0)

    ssm = jnp.where(sv > 0.5, ss, -1e30)

    def nms_body(t, carry):
        keep, cnt, processed, ox1, oy1, ox2, oy2, osc = carry
        mkey = jnp.where(processed > 0.5, _NEG_INF, ssm)
        m = jnp.max(mkey)
        j = jnp.max(jnp.where(mkey == m, lane, -1))
        tm = lane == j
        processed = jnp.where(tm, 1.0, processed)
        x1j = jnp.sum(jnp.where(tm, sx1, 0.0))
        y1j = jnp.sum(jnp.where(tm, sy1, 0.0))
        x2j = jnp.sum(jnp.where(tm, sx2, 0.0))
        y2j = jnp.sum(jnp.where(tm, sy2, 0.0))
        sj = jnp.sum(jnp.where(tm, ss, 0.0))
        vj = jnp.sum(jnp.where(tm, sv, 0.0))
        aj = (x2j - x1j + 1.0) * (y2j - y1j + 1.0)
        xx1 = jnp.maximum(x1j, sx1)
        yy1 = jnp.maximum(y1j, sy1)
        xx2 = jnp.minimum(x2j, sx2)
        yy2 = jnp.minimum(y2j, sy2)
        w_ = jnp.maximum(0.0, xx2 - xx1 + 1.0)
        h_ = jnp.maximum(0.0, yy2 - yy1 + 1.0)
        inter = w_ * h_
        ovr = inter / (aj + areas - inter)
        supp = jnp.max(jnp.where(keep > 0.5, ovr, 0.0)) > _NMS_THRESH
        keepj = (vj > 0.5) & jnp.logical_not(supp)
        keep = jnp.where(tm & keepj, 1.0, keep)
        cm = (lane == cnt) & keepj
        return (keep, cnt + keepj.astype(jnp.int32), processed,
                jnp.where(cm, x1j, ox1), jnp.where(cm, y1j, oy1),
                jnp.where(cm, x2j, ox2), jnp.where(cm, y2j, oy2),
                jnp.where(cm, sj, osc))

    processed0 = (lane >= _PRE_NMS_TOPN).astype(jnp.float32)
    _, _, _, ox1, oy1, ox2, oy2, osc = jax.lax.fori_loop(
        0, _PRE_NMS_TOPN, nms_body,
        (zeros, jnp.int32(0), processed0, zeros, zeros, zeros, zeros, zeros))

    out_ref[0] = jnp.concatenate(
        [ox1, oy1, ox2, oy2, osc, jnp.zeros((3, 128), jnp.float32)], axis=0)


def kernel(score, delta, img):
    B = score.shape[0]
    sc = jnp.transpose(score[:, _A:], (0, 2, 3, 1)).reshape(B, _ROWS, 128)
    d = delta.reshape(B, _N, 4)
    dx = d[:, :, 0].reshape(B, _ROWS, 128)
    dy = d[:, :, 1].reshape(B, _ROWS, 128)
    dw = d[:, :, 2].reshape(B, _ROWS, 128)
    dh = d[:, :, 3].reshape(B, _ROWS, 128)
    img_pad = jnp.pad(img.astype(jnp.float32), (0, 125)).reshape(1, 128)

    wa = jnp.asarray(_WA)
    ha = jnp.asarray(_HA)
    cxa = jnp.asarray(_CXA)
    cya = jnp.asarray(_CYA)

    bspec = pl.BlockSpec((1, _ROWS, 128), lambda b: (b, 0, 0))
    cspec = pl.BlockSpec((_ROWS, 128), lambda b: (0, 0))
    out = pl.pallas_call(
        _proposal_kernel,
        grid=(B,),
        in_specs=[bspec, bspec, bspec, bspec, bspec,
                  cspec, cspec, cspec, cspec,
                  pl.BlockSpec((1, 128), lambda b: (0, 0))],
        out_specs=pl.BlockSpec((1, 8, 128), lambda b: (b, 0, 0)),
        out_shape=jax.ShapeDtypeStruct((B, 8, 128), jnp.float32),
        scratch_shapes=[pltpu.VMEM((_ROWS, 128), jnp.float32)] * 7,
    )(sc, dx, dy, dw, dh, wa, ha, cxa, cya, img_pad)

    return jnp.transpose(out[:, :5, :100], (0, 2, 1))
